# baseline (speedup 1.0000x reference)
import functools

import jax
import jax.numpy as jnp
from jax import lax
from jax.experimental import pallas as pl
from jax.experimental.pallas import tpu as pltpu
from jax.experimental.pallas import tpu_sc as plsc

D_MODEL = 64
SCALE = 8.0
LANES = 16

NUM_CORES = 2
NUM_SUBCORES = 16
NUM_WORKERS = NUM_CORES * NUM_SUBCORES

CHUNK = 128
NBUF = 4


def _make_sc_embed(batch: int):
  assert batch % (NUM_WORKERS * CHUNK * NBUF) == 0
  b_per_w = batch // NUM_WORKERS
  n_chunks = b_per_w // CHUNK
  n_outer = n_chunks // NBUF

  mesh = plsc.VectorSubcoreMesh(
      core_axis_name="c", subcore_axis_name="s",
      num_cores=NUM_CORES, num_subcores=NUM_SUBCORES)

  @functools.partial(
      pl.kernel,
      mesh=mesh,
      compiler_params=pltpu.CompilerParams(use_tc_tiling_on_sc=False),
      out_type=jax.ShapeDtypeStruct((batch, D_MODEL), jnp.float32),
      scratch_types=[
          pltpu.VMEM((n_chunks, CHUNK), jnp.int32),
          [pltpu.VMEM_SHARED((NUM_SUBCORES, CHUNK, D_MODEL), jnp.float32)] * NBUF,
          [pltpu.VMEM((CHUNK, D_MODEL), jnp.float32)] * NBUF,
          [pltpu.SemaphoreType.DMA] * NBUF,
          [pltpu.SemaphoreType.DMA] * NBUF,
      ],
  )
  def embed(idx_hbm, table_hbm, out_hbm, idx_v, bufs_in, bufs_out,
            gsems, osems):
    wid = lax.axis_index("s") * NUM_CORES + lax.axis_index("c")
    base = wid * b_per_w

    pltpu.sync_copy(idx_hbm.at[pl.ds(wid * n_chunks, n_chunks)], idx_v)

    sid = lax.axis_index("s")

    def issue_gather(g, b):
      def group16(k, _):
        r0 = k * LANES
        idxv = idx_v[g, pl.ds(r0, LANES)]
        for l in range(LANES):
          row = idxv[l]
          pltpu.async_copy(
              table_hbm.at[pl.ds(row, 1)],
              bufs_in[b].at[sid, pl.ds(r0 + l, 1)], gsems[b])
        return _

      lax.fori_loop(0, CHUNK // LANES, group16, None)

    def wait_gather(b):
      pltpu.make_async_copy(
          table_hbm.at[pl.ds(0, CHUNK)], bufs_in[b].at[sid], gsems[b]).wait()

    def issue_out(g, b):
      pltpu.async_copy(
          bufs_out[b], out_hbm.at[pl.ds(base + g * CHUNK, CHUNK)], osems[b])

    def wait_out(b):
      pltpu.make_async_copy(
          bufs_out[b], out_hbm.at[pl.ds(0, CHUNK)], osems[b]).wait()

    def scale(b):
      pltpu.sync_copy(bufs_in[b].at[sid], bufs_out[b])
      src, dst = bufs_out[b], bufs_out[b]

      def rows4(r4, _):
        r = r4 * 4
        for dr in range(4):
          for j in range(D_MODEL // LANES):
            sl = pl.ds(j * LANES, LANES)
            dst[r + dr, sl] = src[r + dr, sl] * SCALE
        return _

      lax.fori_loop(0, CHUNK // 4, rows4, None)

    def process(g, b, first, last):
      wait_gather(b)
      if not first:
        wait_out(b)
      scale(b)
      issue_out(g, b)
      if not last:
        issue_gather(g + NBUF, b)

    for b in range(NBUF):
      issue_gather(b, b)
    for b in range(NBUF):
      process(b, b, first=True, last=False)

    def outer(t, _):
      for b in range(NBUF):
        process(t * NBUF + b, b, first=False, last=False)
      return _

    lax.fori_loop(1, n_outer - 1, outer, None)

    for b in range(NBUF):
      process((n_outer - 1) * NBUF + b, b, first=False, last=True)
    for b in range(NBUF):
      wait_out(b)

  return embed


def kernel(x, table):
  batch = x.shape[0] * x.shape[1]
  flat_idx = x.reshape(batch // CHUNK, CHUNK).astype(jnp.int32)
  out = _make_sc_embed(batch)(flat_idx, table)
  return out.reshape(x.shape[0], x.shape[1], D_MODEL)


# speedup vs baseline: 2.1955x; 2.1955x over previous
import functools

import jax
import jax.numpy as jnp
from jax import lax
from jax.experimental import pallas as pl
from jax.experimental.pallas import tpu as pltpu
from jax.experimental.pallas import tpu_sc as plsc

D_MODEL = 64
SCALE = 8.0
LANES = 16

NUM_CORES = 2
NUM_SUBCORES = 16
NUM_WORKERS = NUM_CORES * NUM_SUBCORES

CHUNK = 128
NBUF = 4


def _make_sc_embed(batch: int):
  assert batch % (NUM_WORKERS * CHUNK * NBUF) == 0
  b_per_w = batch // NUM_WORKERS
  n_chunks = b_per_w // CHUNK
  n_outer = n_chunks // NBUF

  mesh = plsc.VectorSubcoreMesh(
      core_axis_name="c", subcore_axis_name="s",
      num_cores=NUM_CORES, num_subcores=NUM_SUBCORES)

  @functools.partial(
      pl.kernel,
      mesh=mesh,
      compiler_params=pltpu.CompilerParams(use_tc_tiling_on_sc=False),
      out_type=jax.ShapeDtypeStruct((batch, D_MODEL), jnp.float32),
      scratch_types=[
          pltpu.VMEM((n_chunks, CHUNK), jnp.int32),
          [pltpu.VMEM((CHUNK, D_MODEL), jnp.float32)] * NBUF,
          [pltpu.VMEM((CHUNK, D_MODEL), jnp.float32)] * NBUF,
          [pltpu.SemaphoreType.DMA] * NBUF,
          [pltpu.SemaphoreType.DMA] * NBUF,
      ],
  )
  def embed(idx_hbm, table_hbm, out_hbm, idx_v, bufs_in, bufs_out,
            gsems, osems):
    wid = lax.axis_index("s") * NUM_CORES + lax.axis_index("c")
    base = wid * b_per_w

    pltpu.sync_copy(idx_hbm.at[pl.ds(wid * n_chunks, n_chunks)], idx_v)

    def issue_gather(g, b):
      pltpu.async_copy(table_hbm.at[idx_v.at[g]], bufs_in[b], gsems[b])

    def wait_gather(b):
      pltpu.make_async_copy(
          table_hbm.at[idx_v.at[0]], bufs_in[b], gsems[b]).wait()

    def issue_out(g, b):
      pltpu.async_copy(
          bufs_out[b], out_hbm.at[pl.ds(base + g * CHUNK, CHUNK)], osems[b])

    def wait_out(b):
      pltpu.make_async_copy(
          bufs_out[b], out_hbm.at[pl.ds(0, CHUNK)], osems[b]).wait()

    def scale(b):
      src, dst = bufs_in[b], bufs_out[b]

      def rows4(r4, _):
        r = r4 * 4
        for dr in range(4):
          for j in range(D_MODEL // LANES):
            sl = pl.ds(j * LANES, LANES)
            dst[r + dr, sl] = src[r + dr, sl] * SCALE
        return _

      lax.fori_loop(0, CHUNK // 4, rows4, None)

    def process(g, b, first, last):
      wait_gather(b)
      if not first:
        wait_out(b)
      scale(b)
      issue_out(g, b)
      if not last:
        issue_gather(g + NBUF, b)

    for b in range(NBUF):
      issue_gather(b, b)
    for b in range(NBUF):
      process(b, b, first=True, last=False)

    def outer(t, _):
      for b in range(NBUF):
        process(t * NBUF + b, b, first=False, last=False)
      return _

    lax.fori_loop(1, n_outer - 1, outer, None)

    for b in range(NBUF):
      process((n_outer - 1) * NBUF + b, b, first=False, last=True)
    for b in range(NBUF):
      wait_out(b)

  return embed


def kernel(x, table):
  batch = x.shape[0] * x.shape[1]
  flat_idx = x.reshape(batch // CHUNK, CHUNK).astype(jnp.int32)
  out = _make_sc_embed(batch)(flat_idx, table)
  return out.reshape(x.shape[0], x.shape[1], D_MODEL)
